# baseline (speedup 1.0000x reference)
import functools

import jax
import jax.numpy as jnp
from jax import lax
from jax.experimental import pallas as pl
from jax.experimental.pallas import tpu as pltpu
from jax.experimental.pallas import tpu_sc as plsc

N = 10000
D = 128
H = 128
OUT = 8
E = 320000

NPAD = 10240
EP = 327680
CH = 128
PT = 80
ROWS2D = EP // CH
TRASH = 10200

PT0 = 80
PT1 = 80

NC = 2
NS = 16
SLICE = NPAD // NS
ZCH = 128

_MESH = dict(core_axis_name="c", subcore_axis_name="s")


@functools.lru_cache(maxsize=None)
def _get_deg_pass():
    @functools.partial(
        pl.kernel,
        out_type=jax.ShapeDtypeStruct((NC, NPAD, D), jnp.float32),
        mesh=plsc.VectorSubcoreMesh(**_MESH),
        scratch_types=[
            pltpu.VMEM((PT, CH), jnp.int32),
            pltpu.VMEM((ZCH, D), jnp.float32),
            pltpu.VMEM((ZCH, D), jnp.float32),
            pltpu.VMEM_SHARED((NPAD, D), jnp.float32),
        ],
    )
    def _deg_pass(dst_hbm, ones_hbm, zeros_hbm, out_hbm, idx_d, ones_v, stage_v, table_sh):
        cid = lax.axis_index("c")
        sid = lax.axis_index("s")
        wid = sid * NC + cid
        base = sid * SLICE
        pltpu.sync_copy(zeros_hbm, stage_v)
        for k in range(SLICE // ZCH):
            pltpu.sync_copy(stage_v, table_sh.at[pl.ds(base + k * ZCH, ZCH)])
        pltpu.sync_copy(ones_hbm, ones_v)
        pltpu.sync_copy(dst_hbm.at[pl.ds(wid * PT, PT)], idx_d)
        plsc.subcore_barrier()

        def body(j, carry):
            pltpu.sync_copy(ones_v, table_sh.at[idx_d.at[j]], add=True)
            return carry

        lax.fori_loop(0, PT, body, 0)
        plsc.subcore_barrier()
        for k in range(SLICE // ZCH):
            pltpu.sync_copy(table_sh.at[pl.ds(base + k * ZCH, ZCH)], stage_v)
            pltpu.sync_copy(stage_v, out_hbm.at[cid, pl.ds(base + k * ZCH, ZCH)])

    return _deg_pass


@functools.lru_cache(maxsize=None)
def _get_edge_pass():
    @functools.partial(
        pl.kernel,
        out_type=jax.ShapeDtypeStruct((NC, NPAD, D), jnp.float32),
        mesh=plsc.VectorSubcoreMesh(**_MESH),
        scratch_types=[
            pltpu.VMEM((2, CH), jnp.int32),
            pltpu.VMEM((2, CH), jnp.int32),
            pltpu.VMEM((2, CH), jnp.int32),
            pltpu.VMEM((2, CH), jnp.int32),
            pltpu.VMEM((2 * CH, D), jnp.float32),
            pltpu.VMEM_SHARED((NPAD, D), jnp.float32),
            pltpu.SemaphoreType.DMA,
            pltpu.SemaphoreType.DMA,
            pltpu.SemaphoreType.DMA,
            pltpu.SemaphoreType.DMA,
            pltpu.SemaphoreType.DMA,
            pltpu.SemaphoreType.DMA,
            pltpu.SemaphoreType.DMA,
            pltpu.SemaphoreType.DMA,
        ],
    )
    def _edge_pass(g_hbm, edges_hbm, zeros_hbm, out_hbm, ib0, ib1, ib2, ib3, rows_v,
                   acc_sh, gs0, gs1, ss0, ss1, is0, is1, is2, is3):
        ibufs = [ib0, ib1, ib2, ib3]
        gsems = [gs0, gs1]
        ssems = [ss0, ss1]
        isems = [is0, is1, is2, is3]
        rows = [rows_v.at[pl.ds(b * CH, CH)] for b in range(2)]
        cid = lax.axis_index("c")
        sid = lax.axis_index("s")
        base = sid * SLICE
        nch = jnp.where(cid == 0, PT0, PT1)
        row0 = jnp.where(cid == 0, sid * PT0, NS * PT0 + sid * PT1)
        pltpu.sync_copy(zeros_hbm, rows[0])
        for k in range(SLICE // ZCH):
            pltpu.sync_copy(rows[0], acc_sh.at[pl.ds(base + k * ZCH, ZCH)])
        plsc.subcore_barrier()

        def idx_load(j, q):
            pltpu.async_copy(edges_hbm.at[row0 + j], ibufs[q], isems[q])

        def idx_wait(j, q):
            pltpu.make_async_copy(edges_hbm.at[row0 + j], ibufs[q], isems[q]).wait()

        def gather(q, b):
            pltpu.async_copy(g_hbm.at[ibufs[q].at[0]], rows[b], gsems[b])

        def gather_wait(q, b):
            pltpu.make_async_copy(g_hbm.at[ibufs[q].at[0]], rows[b], gsems[b]).wait()

        def scat(q, b):
            pltpu.async_copy(rows[b], acc_sh.at[ibufs[q].at[1]], ssems[b], add=True)

        def scat_wait(q, b):
            pltpu.make_async_copy(rows[b], acc_sh.at[ibufs[q].at[1]], ssems[b]).wait()

        @pl.when(nch > 0)
        def _():
            idx_load(0, 0)

            @pl.when(nch > 1)
            def _():
                idx_load(1, 1)

            idx_wait(0, 0)
            gather(0, 0)

        def body(jj, carry):
            for u in range(4):
                j = jj * 4 + u
                b = u % 2
                nb = 1 - b
                q = u
                qn = (u + 1) % 4
                ql = (u + 2) % 4
                qp = (u + 3) % 4

                @pl.when(j < nch)
                def _():
                    gather_wait(q, b)
                    scat(q, b)

                    @pl.when(j >= 1)
                    def _():
                        scat_wait(qp, nb)

                    @pl.when(j + 1 < nch)
                    def _():
                        idx_wait(j + 1, qn)
                        gather(qn, nb)

                    @pl.when(j + 2 < nch)
                    def _():
                        idx_load(j + 2, ql)
            return carry

        lax.fori_loop(0, max(PT0, PT1) // 4, body, 0)

        if PT0 > 0:
            @pl.when(cid == 0)
            def _():
                scat_wait((PT0 - 1) % 4, (PT0 - 1) % 2)

        if PT1 > 0:
            @pl.when(cid == 1)
            def _():
                scat_wait((PT1 - 1) % 4, (PT1 - 1) % 2)

        plsc.subcore_barrier()
        for k in range(SLICE // ZCH):
            pltpu.sync_copy(acc_sh.at[pl.ds(base + k * ZCH, ZCH)], rows[0])
            pltpu.sync_copy(rows[0], out_hbm.at[cid, pl.ds(base + k * ZCH, ZCH)])

    return _edge_pass


BR = 512
GRID = NPAD // BR


def _dinv_block(d0_ref, d1_ref):
    deg = d0_ref[:, :1] + d1_ref[:, :1] + 1.0
    return lax.rsqrt(deg)


def _tc1(x, W1, d0, d1):
    def body(x_ref, w_ref, d0_ref, d1_ref, g_ref):
        dinv = _dinv_block(d0_ref, d1_ref)
        h = jnp.dot(x_ref[...], w_ref[...], preferred_element_type=jnp.float32)
        g_ref[...] = h * dinv

    return pl.pallas_call(
        body,
        grid=(GRID,),
        in_specs=[
            pl.BlockSpec((BR, D), lambda i: (i, 0)),
            pl.BlockSpec((D, H), lambda i: (0, 0)),
            pl.BlockSpec((BR, D), lambda i: (i, 0)),
            pl.BlockSpec((BR, D), lambda i: (i, 0)),
        ],
        out_specs=pl.BlockSpec((BR, H), lambda i: (i, 0)),
        out_shape=jax.ShapeDtypeStruct((NPAD, H), jnp.float32),
    )(x, W1, d0, d1)


def _tc2(p0, p1, g, d0, d1, b, W):
    def body(p0_ref, p1_ref, g_ref, d0_ref, d1_ref, b_ref, w_ref, o_ref):
        dinv = _dinv_block(d0_ref, d1_ref)
        t = dinv * (p0_ref[...] + p1_ref[...] + g_ref[...]) + b_ref[...]
        t = jnp.maximum(t, 0.0)
        h = jnp.dot(t, w_ref[...], preferred_element_type=jnp.float32)
        o_ref[...] = h * dinv

    return pl.pallas_call(
        body,
        grid=(GRID,),
        in_specs=[
            pl.BlockSpec((BR, H), lambda i: (i, 0)),
            pl.BlockSpec((BR, H), lambda i: (i, 0)),
            pl.BlockSpec((BR, H), lambda i: (i, 0)),
            pl.BlockSpec((BR, D), lambda i: (i, 0)),
            pl.BlockSpec((BR, D), lambda i: (i, 0)),
            pl.BlockSpec((1, H), lambda i: (0, 0)),
            pl.BlockSpec((H, H), lambda i: (0, 0)),
        ],
        out_specs=pl.BlockSpec((BR, H), lambda i: (i, 0)),
        out_shape=jax.ShapeDtypeStruct((NPAD, H), jnp.float32),
    )(p0, p1, g, d0, d1, b, W)


def _tc3(q0, q1, g, d0, d1, b2, Wq1, bq1, Wq2p, bq2p):
    def body(q0_ref, q1_ref, g_ref, d0_ref, d1_ref, b2_ref, wq1_ref, bq1_ref,
             wq2_ref, bq2_ref, o_ref):
        dinv = _dinv_block(d0_ref, d1_ref)
        t = dinv * (q0_ref[...] + q1_ref[...] + g_ref[...]) + b2_ref[...]
        t = jnp.maximum(t, 0.0)
        z = jnp.dot(t, wq1_ref[...], preferred_element_type=jnp.float32) + bq1_ref[...]
        z = jnp.maximum(z, 0.0)
        o_ref[...] = jnp.dot(z, wq2_ref[...], preferred_element_type=jnp.float32) + bq2_ref[...]

    return pl.pallas_call(
        body,
        grid=(GRID,),
        in_specs=[
            pl.BlockSpec((BR, H), lambda i: (i, 0)),
            pl.BlockSpec((BR, H), lambda i: (i, 0)),
            pl.BlockSpec((BR, H), lambda i: (i, 0)),
            pl.BlockSpec((BR, D), lambda i: (i, 0)),
            pl.BlockSpec((BR, D), lambda i: (i, 0)),
            pl.BlockSpec((1, H), lambda i: (0, 0)),
            pl.BlockSpec((H, H), lambda i: (0, 0)),
            pl.BlockSpec((1, H), lambda i: (0, 0)),
            pl.BlockSpec((H, 128), lambda i: (0, 0)),
            pl.BlockSpec((1, 128), lambda i: (0, 0)),
        ],
        out_specs=pl.BlockSpec((BR, 128), lambda i: (i, 0)),
        out_shape=jax.ShapeDtypeStruct((NPAD, 128), jnp.float32),
    )(q0, q1, g, d0, d1, b2, Wq1, bq1, Wq2p, bq2p)


def kernel(x, edge_index, W1, b1, W2, b2, Wq1, bq1, Wq2, bq2):
    f32 = jnp.float32
    src = edge_index[0]
    dst = edge_index[1]
    pad_e = EP - E
    pad_ar = jnp.arange(pad_e, dtype=jnp.int32)
    src_pad = pad_ar % N
    dst_pad = N + (pad_ar % (NPAD - N))
    src2d = jnp.concatenate([src, src_pad]).reshape(ROWS2D, CH)
    dst2d = jnp.concatenate([dst, dst_pad]).reshape(ROWS2D, CH)
    edges3d = jnp.stack([src2d, dst2d], axis=1)
    x_pad = jnp.concatenate([x, jnp.zeros((NPAD - N, D), f32)], axis=0)
    ones128 = jnp.ones((ZCH, D), f32)
    zeros128 = jnp.zeros((ZCH, D), f32)

    deg_pass = _get_deg_pass()
    edge_pass = _get_edge_pass()
    degp = deg_pass(dst2d, ones128, zeros128)
    d0, d1 = degp[0], degp[1]

    g1 = _tc1(x_pad, W1, d0, d1)
    p = edge_pass(g1, edges3d, zeros128)
    g2 = _tc2(p[0], p[1], g1, d0, d1, b1.reshape(1, H), W2)
    q = edge_pass(g2, edges3d, zeros128)

    Wq2p = jnp.zeros((H, 128), f32).at[:, :OUT].set(Wq2)
    bq2p = jnp.zeros((1, 128), f32).at[0, :OUT].set(bq2)
    out = _tc3(q[0], q[1], g2, d0, d1, b2.reshape(1, H), Wq1,
               bq1.reshape(1, H), Wq2p, bq2p)
    return out[:N, :OUT]


# speedup vs baseline: 1.1044x; 1.1044x over previous
import functools

import jax
import jax.numpy as jnp
from jax import lax
from jax.experimental import pallas as pl
from jax.experimental.pallas import tpu as pltpu
from jax.experimental.pallas import tpu_sc as plsc

N = 10000
D = 128
H = 128
OUT = 8
E = 320000

NPAD = 10240
EP = 327680
CH = 128
PT = 80
ROWS2D = EP // CH
TRASH = 10200

PT0 = 80
PT1 = 80

NC = 2
NS = 16
SLICE = NPAD // NS
ZCH = 128

_MESH = dict(core_axis_name="c", subcore_axis_name="s")


@functools.lru_cache(maxsize=None)
def _get_deg_pass():
    @functools.partial(
        pl.kernel,
        out_type=jax.ShapeDtypeStruct((NC, NPAD, D), jnp.float32),
        mesh=plsc.VectorSubcoreMesh(**_MESH),
        scratch_types=[
            pltpu.VMEM((PT, CH), jnp.int32),
            pltpu.VMEM((ZCH, D), jnp.float32),
            pltpu.VMEM((ZCH, D), jnp.float32),
            pltpu.VMEM_SHARED((NPAD, D), jnp.float32),
        ],
    )
    def _deg_pass(e3_hbm, ones_hbm, zeros_hbm, out_hbm, idx_d, ones_v, stage_v, table_sh):
        cid = lax.axis_index("c")
        sid = lax.axis_index("s")
        wid = sid * NC + cid
        base = sid * SLICE
        pltpu.sync_copy(zeros_hbm, stage_v)
        for k in range(SLICE // ZCH):
            pltpu.sync_copy(stage_v, table_sh.at[pl.ds(base + k * ZCH, ZCH)])
        pltpu.sync_copy(ones_hbm, ones_v)
        pltpu.sync_copy(e3_hbm.at[1, pl.ds(wid * PT, PT)], idx_d)
        plsc.subcore_barrier()

        def body(j, carry):
            pltpu.sync_copy(ones_v, table_sh.at[idx_d.at[j]], add=True)
            return carry

        lax.fori_loop(0, PT, body, 0)
        plsc.subcore_barrier()
        for k in range(SLICE // ZCH):
            pltpu.sync_copy(table_sh.at[pl.ds(base + k * ZCH, ZCH)], stage_v)
            pltpu.sync_copy(stage_v, out_hbm.at[cid, pl.ds(base + k * ZCH, ZCH)])

    return _deg_pass


@functools.lru_cache(maxsize=None)
def _get_edge_pass():
    @functools.partial(
        pl.kernel,
        out_type=jax.ShapeDtypeStruct((NC, NPAD, D), jnp.float32),
        mesh=plsc.VectorSubcoreMesh(**_MESH),
        scratch_types=[
            pltpu.VMEM((2, CH), jnp.int32),
            pltpu.VMEM((2, CH), jnp.int32),
            pltpu.VMEM((2 * CH, D), jnp.float32),
            pltpu.VMEM_SHARED((NPAD, D), jnp.float32),
            pltpu.SemaphoreType.DMA,
            pltpu.SemaphoreType.DMA,
            pltpu.SemaphoreType.DMA,
            pltpu.SemaphoreType.DMA,
        ],
    )
    def _edge_pass(g_hbm, edges_hbm, zeros_hbm, out_hbm, ib0, ib1, rows_v,
                   acc_sh, gs0, gs1, is0, is1):
        ibufs = [ib0, ib1]
        gsems = [gs0, gs1]
        isems = [is0, is1]
        rows = [rows_v.at[pl.ds(b * CH, CH)] for b in range(2)]
        cid = lax.axis_index("c")
        sid = lax.axis_index("s")
        base = sid * SLICE
        nch = jnp.where(cid == 0, PT0, PT1)
        row0 = jnp.where(cid == 0, sid * PT0, NS * PT0 + sid * PT1)
        pltpu.sync_copy(zeros_hbm, rows[0])
        for k in range(SLICE // ZCH):
            pltpu.sync_copy(rows[0], acc_sh.at[pl.ds(base + k * ZCH, ZCH)])
        plsc.subcore_barrier()

        def idx_load(j, b):
            pltpu.async_copy(edges_hbm.at[:, row0 + j], ibufs[b], isems[b])

        def idx_wait(j, b):
            pltpu.make_async_copy(edges_hbm.at[:, row0 + j], ibufs[b], isems[b]).wait()

        def gather(b):
            pltpu.async_copy(g_hbm.at[ibufs[b].at[0]], rows[b], gsems[b])

        def gather_wait(b):
            pltpu.make_async_copy(g_hbm.at[ibufs[b].at[0]], rows[b], gsems[b]).wait()

        @pl.when(nch > 0)
        def _():
            idx_load(0, 0)

            @pl.when(nch > 1)
            def _():
                idx_load(1, 1)

            idx_wait(0, 0)
            gather(0)

        def body(jj, carry):
            for b in range(2):
                j = jj * 2 + b
                nb = 1 - b

                @pl.when(j < nch)
                def _():
                    @pl.when(j + 1 < nch)
                    def _():
                        idx_wait(j + 1, nb)
                        gather(nb)

                    gather_wait(b)
                    pltpu.sync_copy(rows[b], acc_sh.at[ibufs[b].at[1]], add=True)

                    @pl.when(j + 2 < nch)
                    def _():
                        idx_load(j + 2, b)
            return carry

        lax.fori_loop(0, max(PT0, PT1) // 2, body, 0)
        plsc.subcore_barrier()
        for k in range(SLICE // ZCH):
            pltpu.sync_copy(acc_sh.at[pl.ds(base + k * ZCH, ZCH)], rows[0])
            pltpu.sync_copy(rows[0], out_hbm.at[cid, pl.ds(base + k * ZCH, ZCH)])

    return _edge_pass


BR = 512
GRID = NPAD // BR


def _dinv3(da_ref, db_ref):
    deg = da_ref[0][:, :1] + db_ref[0][:, :1] + 1.0
    return lax.rsqrt(deg)


def _pl3(plane):
    return pl.BlockSpec((1, BR, D), lambda i, _p=plane: (_p, i, 0))


def _tc1(x, W1, degp):
    def body(x_ref, w_ref, da_ref, db_ref, g_ref):
        dinv = _dinv3(da_ref, db_ref)
        h = jnp.dot(x_ref[...], w_ref[...], preferred_element_type=jnp.float32)
        g_ref[...] = h * dinv

    return pl.pallas_call(
        body,
        grid=(GRID,),
        in_specs=[
            pl.BlockSpec((BR, D), lambda i: (i, 0)),
            pl.BlockSpec((D, H), lambda i: (0, 0)),
            _pl3(0),
            _pl3(1),
        ],
        out_specs=pl.BlockSpec((BR, H), lambda i: (i, 0)),
        out_shape=jax.ShapeDtypeStruct((NPAD, H), jnp.float32),
    )(x, W1, degp, degp)


def _tc2(p, g, degp, b, W):
    def body(pa_ref, pb_ref, g_ref, da_ref, db_ref, b_ref, w_ref, o_ref):
        dinv = _dinv3(da_ref, db_ref)
        t = dinv * (pa_ref[0] + pb_ref[0] + g_ref[...]) + b_ref[...]
        t = jnp.maximum(t, 0.0)
        h = jnp.dot(t, w_ref[...], preferred_element_type=jnp.float32)
        o_ref[...] = h * dinv

    return pl.pallas_call(
        body,
        grid=(GRID,),
        in_specs=[
            _pl3(0),
            _pl3(1),
            pl.BlockSpec((BR, H), lambda i: (i, 0)),
            _pl3(0),
            _pl3(1),
            pl.BlockSpec((1, H), lambda i: (0, 0)),
            pl.BlockSpec((H, H), lambda i: (0, 0)),
        ],
        out_specs=pl.BlockSpec((BR, H), lambda i: (i, 0)),
        out_shape=jax.ShapeDtypeStruct((NPAD, H), jnp.float32),
    )(p, p, g, degp, degp, b, W)


def _tc3(q, g, degp, b2, Wq1, bq1, Wq2p, bq2p):
    def body(qa_ref, qb_ref, g_ref, da_ref, db_ref, b2_ref, wq1_ref, bq1_ref,
             wq2_ref, bq2_ref, o_ref):
        dinv = _dinv3(da_ref, db_ref)
        t = dinv * (qa_ref[0] + qb_ref[0] + g_ref[...]) + b2_ref[...]
        t = jnp.maximum(t, 0.0)
        z = jnp.dot(t, wq1_ref[...], preferred_element_type=jnp.float32) + bq1_ref[...]
        z = jnp.maximum(z, 0.0)
        o_ref[...] = jnp.dot(z, wq2_ref[...], preferred_element_type=jnp.float32) + bq2_ref[...]

    return pl.pallas_call(
        body,
        grid=(GRID,),
        in_specs=[
            _pl3(0),
            _pl3(1),
            pl.BlockSpec((BR, H), lambda i: (i, 0)),
            _pl3(0),
            _pl3(1),
            pl.BlockSpec((1, H), lambda i: (0, 0)),
            pl.BlockSpec((H, H), lambda i: (0, 0)),
            pl.BlockSpec((1, H), lambda i: (0, 0)),
            pl.BlockSpec((H, 128), lambda i: (0, 0)),
            pl.BlockSpec((1, 128), lambda i: (0, 0)),
        ],
        out_specs=pl.BlockSpec((BR, 128), lambda i: (i, 0)),
        out_shape=jax.ShapeDtypeStruct((NPAD, 128), jnp.float32),
    )(q, q, g, degp, degp, b2, Wq1, bq1, Wq2p, bq2p)


def kernel(x, edge_index, W1, b1, W2, b2, Wq1, bq1, Wq2, bq2):
    f32 = jnp.float32
    pad_e = EP - E
    pad_ar = jnp.arange(pad_e, dtype=jnp.int32)
    pad2 = jnp.stack([pad_ar % N, N + (pad_ar % (NPAD - N))])
    e3 = jnp.concatenate([edge_index, pad2], axis=1).reshape(2, ROWS2D, CH)
    x_pad = jnp.concatenate([x, jnp.zeros((NPAD - N, D), f32)], axis=0)
    ones128 = jnp.ones((ZCH, D), f32)
    zeros128 = jnp.zeros((ZCH, D), f32)

    deg_pass = _get_deg_pass()
    edge_pass = _get_edge_pass()
    degp = deg_pass(e3, ones128, zeros128)

    g1 = _tc1(x_pad, W1, degp)
    p = edge_pass(g1, e3, zeros128)
    g2 = _tc2(p, g1, degp, b1.reshape(1, H), W2)
    q = edge_pass(g2, e3, zeros128)

    Wq2p = jnp.zeros((H, 128), f32).at[:, :OUT].set(Wq2)
    bq2p = jnp.zeros((1, 128), f32).at[0, :OUT].set(bq2)
    out = _tc3(q, g2, degp, b2.reshape(1, H), Wq1,
               bq1.reshape(1, H), Wq2p, bq2p)
    return out[:N, :OUT]


# speedup vs baseline: 1.1045x; 1.0001x over previous
import functools

import jax
import jax.numpy as jnp
from jax import lax
from jax.experimental import pallas as pl
from jax.experimental.pallas import tpu as pltpu
from jax.experimental.pallas import tpu_sc as plsc

N = 10000
D = 128
H = 128
OUT = 8
E = 320000

NPAD = 10240
EP = 327680
CH = 128
PT = 80
ROWS2D = EP // CH

PT0 = 80
PT1 = 80

NC = 2
NS = 16
SLICE = NPAD // NS
ZCH = 128

_MESH = dict(core_axis_name="c", subcore_axis_name="s")


@functools.lru_cache(maxsize=None)
def _get_deg_pass():
    @functools.partial(
        pl.kernel,
        out_type=jax.ShapeDtypeStruct((NC, NPAD, D), jnp.float32),
        mesh=plsc.VectorSubcoreMesh(**_MESH),
        scratch_types=[
            pltpu.VMEM((PT, CH), jnp.int32),
            pltpu.VMEM((ZCH, D), jnp.float32),
            pltpu.VMEM((ZCH, D), jnp.float32),
            pltpu.VMEM_SHARED((NPAD, D), jnp.float32),
        ],
    )
    def _deg_pass(e3_hbm, ones_hbm, zeros_hbm, out_hbm, idx_d, ones_v, stage_v, table_sh):
        cid = lax.axis_index("c")
        sid = lax.axis_index("s")
        wid = sid * NC + cid
        base = sid * SLICE
        pltpu.sync_copy(zeros_hbm, stage_v)
        for k in range(SLICE // ZCH):
            pltpu.sync_copy(stage_v, table_sh.at[pl.ds(base + k * ZCH, ZCH)])
        pltpu.sync_copy(ones_hbm, ones_v)
        pltpu.sync_copy(e3_hbm.at[1, pl.ds(wid * PT, PT)], idx_d)
        plsc.subcore_barrier()

        def body(j, carry):
            pltpu.sync_copy(ones_v, table_sh.at[idx_d.at[j]], add=True)
            return carry

        lax.fori_loop(0, PT, body, 0)
        plsc.subcore_barrier()
        for k in range(SLICE // ZCH):
            pltpu.sync_copy(table_sh.at[pl.ds(base + k * ZCH, ZCH)], stage_v)
            pltpu.sync_copy(stage_v, out_hbm.at[cid, pl.ds(base + k * ZCH, ZCH)])

    return _deg_pass


@functools.lru_cache(maxsize=None)
def _get_edge_pass():
    @functools.partial(
        pl.kernel,
        out_type=jax.ShapeDtypeStruct((NC, NPAD, D), jnp.float32),
        mesh=plsc.VectorSubcoreMesh(**_MESH),
        scratch_types=[
            pltpu.VMEM((2, CH), jnp.int32),
            pltpu.VMEM((2, CH), jnp.int32),
            pltpu.VMEM((2 * CH, D), jnp.float32),
            pltpu.VMEM_SHARED((NPAD, D), jnp.float32),
            pltpu.SemaphoreType.DMA,
            pltpu.SemaphoreType.DMA,
            pltpu.SemaphoreType.DMA,
            pltpu.SemaphoreType.DMA,
        ],
    )
    def _edge_pass(g_hbm, edges_hbm, zeros_hbm, out_hbm, ib0, ib1, rows_v,
                   acc_sh, gs0, gs1, is0, is1):
        ibufs = [ib0, ib1]
        gsems = [gs0, gs1]
        isems = [is0, is1]
        rows = [rows_v.at[pl.ds(b * CH, CH)] for b in range(2)]
        cid = lax.axis_index("c")
        sid = lax.axis_index("s")
        base = sid * SLICE
        nch = jnp.where(cid == 0, PT0, PT1)
        row0 = jnp.where(cid == 0, sid * PT0, NS * PT0 + sid * PT1)
        pltpu.sync_copy(zeros_hbm, rows[0])
        for k in range(SLICE // ZCH):
            pltpu.sync_copy(rows[0], acc_sh.at[pl.ds(base + k * ZCH, ZCH)])
        plsc.subcore_barrier()

        def idx_load(j, b):
            pltpu.async_copy(edges_hbm.at[:, row0 + j], ibufs[b], isems[b])

        def idx_wait(j, b):
            pltpu.make_async_copy(edges_hbm.at[:, row0 + j], ibufs[b], isems[b]).wait()

        def gather(b):
            pltpu.async_copy(g_hbm.at[ibufs[b].at[0]], rows[b], gsems[b])

        def gather_wait(b):
            pltpu.make_async_copy(g_hbm.at[ibufs[b].at[0]], rows[b], gsems[b]).wait()

        @pl.when(nch > 0)
        def _():
            idx_load(0, 0)

            @pl.when(nch > 1)
            def _():
                idx_load(1, 1)

            idx_wait(0, 0)
            gather(0)

        def body(jj, carry):
            for b in range(2):
                j = jj * 2 + b
                nb = 1 - b

                @pl.when(j < nch)
                def _():
                    @pl.when(j + 1 < nch)
                    def _():
                        idx_wait(j + 1, nb)
                        gather(nb)

                    gather_wait(b)
                    pltpu.sync_copy(rows[b], acc_sh.at[ibufs[b].at[1]], add=True)

                    @pl.when(j + 2 < nch)
                    def _():
                        idx_load(j + 2, b)
            return carry

        lax.fori_loop(0, max(PT0, PT1) // 2, body, 0)
        plsc.subcore_barrier()
        for k in range(SLICE // ZCH):
            pltpu.sync_copy(acc_sh.at[pl.ds(base + k * ZCH, ZCH)], rows[0])
            pltpu.sync_copy(rows[0], out_hbm.at[cid, pl.ds(base + k * ZCH, ZCH)])

    return _edge_pass


BR = 512
GRID = NPAD // BR


def _dinv3(da_ref, db_ref):
    deg = da_ref[0][:, :1] + db_ref[0][:, :1] + 1.0
    return lax.rsqrt(deg)


def _pl3(plane):
    return pl.BlockSpec((1, BR, D), lambda i, _p=plane: (_p, i, 0))


def _tc1(x, W1, degp):
    def body(x_ref, w_ref, da_ref, db_ref, g_ref):
        dinv = _dinv3(da_ref, db_ref)
        h = jnp.dot(x_ref[...], w_ref[...], preferred_element_type=jnp.float32)
        g_ref[...] = h * dinv

    return pl.pallas_call(
        body,
        grid=(GRID,),
        in_specs=[
            pl.BlockSpec((BR, D), lambda i: (i, 0)),
            pl.BlockSpec((D, H), lambda i: (0, 0)),
            _pl3(0),
            _pl3(1),
        ],
        out_specs=pl.BlockSpec((BR, H), lambda i: (i, 0)),
        out_shape=jax.ShapeDtypeStruct((NPAD, H), jnp.float32),
    )(x, W1, degp, degp)


def _tc2(p, g, degp, b, W):
    def body(pa_ref, pb_ref, g_ref, da_ref, db_ref, b_ref, w_ref, o_ref):
        dinv = _dinv3(da_ref, db_ref)
        t = dinv * (pa_ref[0] + pb_ref[0] + g_ref[...]) + b_ref[...]
        t = jnp.maximum(t, 0.0)
        h = jnp.dot(t, w_ref[...], preferred_element_type=jnp.float32)
        o_ref[...] = h * dinv

    return pl.pallas_call(
        body,
        grid=(GRID,),
        in_specs=[
            _pl3(0),
            _pl3(1),
            pl.BlockSpec((BR, H), lambda i: (i, 0)),
            _pl3(0),
            _pl3(1),
            pl.BlockSpec((1, H), lambda i: (0, 0)),
            pl.BlockSpec((H, H), lambda i: (0, 0)),
        ],
        out_specs=pl.BlockSpec((BR, H), lambda i: (i, 0)),
        out_shape=jax.ShapeDtypeStruct((NPAD, H), jnp.float32),
    )(p, p, g, degp, degp, b, W)


def _tc3(q, g, degp, b2, Wq1, bq1, Wq2p, bq2p):
    def body(qa_ref, qb_ref, g_ref, da_ref, db_ref, b2_ref, wq1_ref, bq1_ref,
             wq2_ref, bq2_ref, o_ref):
        dinv = _dinv3(da_ref, db_ref)
        t = dinv * (qa_ref[0] + qb_ref[0] + g_ref[...]) + b2_ref[...]
        t = jnp.maximum(t, 0.0)
        z = jnp.dot(t, wq1_ref[...], preferred_element_type=jnp.float32) + bq1_ref[...]
        z = jnp.maximum(z, 0.0)
        o_ref[...] = jnp.dot(z, wq2_ref[...], preferred_element_type=jnp.float32) + bq2_ref[...]

    return pl.pallas_call(
        body,
        grid=(GRID,),
        in_specs=[
            _pl3(0),
            _pl3(1),
            pl.BlockSpec((BR, H), lambda i: (i, 0)),
            _pl3(0),
            _pl3(1),
            pl.BlockSpec((1, H), lambda i: (0, 0)),
            pl.BlockSpec((H, H), lambda i: (0, 0)),
            pl.BlockSpec((1, H), lambda i: (0, 0)),
            pl.BlockSpec((H, 128), lambda i: (0, 0)),
            pl.BlockSpec((1, 128), lambda i: (0, 0)),
        ],
        out_specs=pl.BlockSpec((BR, 128), lambda i: (i, 0)),
        out_shape=jax.ShapeDtypeStruct((NPAD, 128), jnp.float32),
    )(q, q, g, degp, degp, b2, Wq1, bq1, Wq2p, bq2p)


def kernel(x, edge_index, W1, b1, W2, b2, Wq1, bq1, Wq2, bq2):
    f32 = jnp.float32
    pad_e = EP - E
    pad_ar = jnp.arange(pad_e, dtype=jnp.int32)
    pad2 = jnp.stack([pad_ar % N, N + (pad_ar % (NPAD - N))])
    e3 = jnp.concatenate([edge_index, pad2], axis=1).reshape(2, ROWS2D, CH)
    x_pad = jnp.concatenate([x, jnp.zeros((NPAD - N, D), f32)], axis=0)
    ones128 = jnp.ones((ZCH, D), f32)
    zeros128 = jnp.zeros((ZCH, D), f32)

    deg_pass = _get_deg_pass()
    edge_pass = _get_edge_pass()
    degp = deg_pass(e3, ones128, zeros128)

    g1 = _tc1(x_pad, W1, degp)
    p = edge_pass(g1, e3, zeros128)
    g2 = _tc2(p, g1, degp, b1.reshape(1, H), W2)
    q = edge_pass(g2, e3, zeros128)

    Wq2p = jnp.zeros((H, 128), f32).at[:, :OUT].set(Wq2)
    bq2p = jnp.zeros((1, 128), f32).at[0, :OUT].set(bq2)
    out = _tc3(q, g2, degp, b2.reshape(1, H), Wq1,
               bq1.reshape(1, H), Wq2p, bq2p)
    return out[:N, :OUT]
